# baseline (speedup 1.0000x reference)
import functools

import jax
import jax.numpy as jnp
from jax import lax
from jax.experimental import pallas as pl
from jax.experimental.pallas import tpu as pltpu
from jax.experimental.pallas import tpu_sc as plsc

N = 10000
E = 160000
D = 256
H = 16

NC = 2
NS = 16
NW = NC * NS

C = 2500
CH = E // C
K = CH // NW
NP = 10240
ACC_ROWS = NP

ROWS_PER_SUB = NP // NS

NWIDE = NP // 8
XPLANES = N // 8

_MESH = plsc.VectorSubcoreMesh(
    core_axis_name="c", subcore_axis_name="s", num_cores=NC, num_subcores=NS
)

_SC_PARAMS = pltpu.CompilerParams(use_tc_tiling_on_sc=False)


@functools.partial(
    pl.kernel,
    out_type=jax.ShapeDtypeStruct((NC, NP, H), jnp.float32),
    mesh=_MESH,
    compiler_params=_SC_PARAMS,
    scratch_types=[
        pltpu.VMEM((K, 2, C), jnp.int32),
        pltpu.VMEM((C, H), jnp.float32),
        pltpu.VMEM((ROWS_PER_SUB, H), jnp.float32),
        pltpu.VMEM_SHARED((ACC_ROWS, H), jnp.float32),
        pltpu.SemaphoreType.DMA,
    ],
)
def _deg_kernel(ei3, zeros16, ones2d, out, eidx, ones_v, stage, dacc, lsem):
    c = lax.axis_index("c")
    s = lax.axis_index("s")
    w = s * NC + c
    cp1 = pltpu.async_copy(ei3.at[pl.ds(w * K, K)], eidx, lsem)
    cp2 = pltpu.async_copy(ones2d, ones_v, lsem)
    pltpu.sync_copy(zeros16.at[pl.ds(s * ROWS_PER_SUB, ROWS_PER_SUB)], stage)
    pltpu.sync_copy(stage, dacc.at[pl.ds(s * ROWS_PER_SUB, ROWS_PER_SUB)])
    cp1.wait()
    cp2.wait()
    plsc.subcore_barrier()
    for j in range(K):
        pltpu.sync_copy(ones_v, dacc.at[eidx.at[j].at[1]], add=True)
    plsc.subcore_barrier()
    pltpu.sync_copy(dacc.at[pl.ds(s * ROWS_PER_SUB, ROWS_PER_SUB)], stage)
    pltpu.sync_copy(stage, out.at[c].at[pl.ds(s * ROWS_PER_SUB, ROWS_PER_SUB)])


@functools.partial(
    pl.kernel,
    out_type=jax.ShapeDtypeStruct((NC, NP, H), jnp.float32),
    mesh=_MESH,
    compiler_params=_SC_PARAMS,
    scratch_types=[
        pltpu.VMEM((K, 2, C), jnp.int32),
        pltpu.VMEM((K, C, H), jnp.float32),
        pltpu.VMEM((ROWS_PER_SUB, H), jnp.float32),
        pltpu.VMEM_SHARED((ACC_ROWS, H), jnp.float32),
        pltpu.SemaphoreType.DMA,
        pltpu.SemaphoreType.DMA,
        pltpu.SemaphoreType.DMA,
        pltpu.SemaphoreType.DMA,
    ],
)
def _scatter_kernel(g, ei3, zeros16, out,
                    eidx, rows, stage, acc, lsem, gsem_a, gsem_b, ssem):
    c = lax.axis_index("c")
    s = lax.axis_index("s")
    w = s * NC + c
    pltpu.sync_copy(ei3.at[pl.ds(w * K, K)], eidx)
    pltpu.async_copy(g.at[eidx.at[0].at[0]], rows.at[0], gsem_a)
    pltpu.async_copy(g.at[eidx.at[1].at[0]], rows.at[1], gsem_b)
    pltpu.sync_copy(zeros16.at[pl.ds(s * ROWS_PER_SUB, ROWS_PER_SUB)], stage)
    pltpu.sync_copy(stage, acc.at[pl.ds(s * ROWS_PER_SUB, ROWS_PER_SUB)])
    plsc.subcore_barrier()
    pltpu.make_async_copy(g.at[eidx.at[0].at[0]], rows.at[0], gsem_a).wait()
    pltpu.async_copy(rows.at[0], acc.at[eidx.at[0].at[1]], ssem, add=True)
    pltpu.make_async_copy(g.at[eidx.at[1].at[0]], rows.at[1], gsem_b).wait()
    pltpu.async_copy(rows.at[1], acc.at[eidx.at[1].at[1]], ssem, add=True)
    pltpu.make_async_copy(rows.at[0], acc.at[eidx.at[0].at[1]], ssem).wait()
    pltpu.make_async_copy(rows.at[1], acc.at[eidx.at[1].at[1]], ssem).wait()
    plsc.subcore_barrier()
    pltpu.sync_copy(acc.at[pl.ds(s * ROWS_PER_SUB, ROWS_PER_SUB)], stage)
    pltpu.sync_copy(stage, out.at[c].at[pl.ds(s * ROWS_PER_SUB, ROWS_PER_SUB)])


BW = 128
GRID = NWIDE // BW
BW2 = 256
GRID2 = NWIDE // BW2


def _dinv_of(degw_ref):
    return lax.rsqrt(degw_ref[0] + degw_ref[1] + 1.0)


def _mm1_body(x3_ref, w_ref, o_ref):
    w = w_ref[...]
    parts = [
        jnp.dot(x3_ref[:, k, :], w, preferred_element_type=jnp.float32)
        for k in range(8)
    ]
    o_ref[...] = jnp.concatenate(parts, axis=1)


_mm1 = pl.pallas_call(
    _mm1_body,
    grid=(GRID,),
    in_specs=[
        pl.BlockSpec((BW, 8, D), lambda i: (i, 0, 0)),
        pl.BlockSpec((D, H), lambda i: (0, 0)),
    ],
    out_specs=pl.BlockSpec((BW, 128), lambda i: (i, 0)),
    out_shape=jax.ShapeDtypeStruct((NWIDE, 128), jnp.float32),
)


def _ew1_body(xw_ref, degw_ref, o_ref):
    o_ref[...] = _dinv_of(degw_ref) * xw_ref[...]


_ew1 = pl.pallas_call(
    _ew1_body,
    grid=(GRID2,),
    in_specs=[
        pl.BlockSpec((BW2, 128), lambda i: (i, 0)),
        pl.BlockSpec((NC, BW2, 128), lambda i: (0, i, 0)),
    ],
    out_specs=pl.BlockSpec((BW2, 128), lambda i: (i, 0)),
    out_shape=jax.ShapeDtypeStruct((NWIDE, 128), jnp.float32),
)


def _l2_body(sp_ref, g_ref, degw_ref, w2k_ref, b1w_ref, o_ref):
    dinv = _dinv_of(degw_ref)
    agg = sp_ref[0] + sp_ref[1] + g_ref[...]
    h1 = jnp.maximum(dinv * agg + b1w_ref[...], 0.0)
    o_ref[...] = dinv * jnp.dot(h1, w2k_ref[...], preferred_element_type=jnp.float32)


_l2 = pl.pallas_call(
    _l2_body,
    grid=(GRID2,),
    in_specs=[
        pl.BlockSpec((NC, BW2, 128), lambda i: (0, i, 0)),
        pl.BlockSpec((BW2, 128), lambda i: (i, 0)),
        pl.BlockSpec((NC, BW2, 128), lambda i: (0, i, 0)),
        pl.BlockSpec((128, 128), lambda i: (0, 0)),
        pl.BlockSpec((1, 128), lambda i: (0, 0)),
    ],
    out_specs=pl.BlockSpec((BW2, 128), lambda i: (i, 0)),
    out_shape=jax.ShapeDtypeStruct((NWIDE, 128), jnp.float32),
)


def _l3_body(sp_ref, g_ref, degw_ref, wo_ref, b2w_ref, bo_ref, o_ref):
    dinv = _dinv_of(degw_ref)
    agg = sp_ref[0] + sp_ref[1] + g_ref[...]
    h2 = jnp.maximum(dinv * agg + b2w_ref[...], 0.0)
    wo = wo_ref[...]
    bo = bo_ref[...]
    for k in range(8):
        hk = h2[:, 16 * k:16 * (k + 1)]
        zk = jnp.dot(hk, wo, preferred_element_type=jnp.float32) + bo
        o_ref[:, k, :] = jax.nn.sigmoid(zk)


_l3 = pl.pallas_call(
    _l3_body,
    grid=(GRID,),
    in_specs=[
        pl.BlockSpec((NC, BW, 128), lambda i: (0, i, 0)),
        pl.BlockSpec((BW, 128), lambda i: (i, 0)),
        pl.BlockSpec((NC, BW, 128), lambda i: (0, i, 0)),
        pl.BlockSpec((H, D), lambda i: (0, 0)),
        pl.BlockSpec((1, 128), lambda i: (0, 0)),
        pl.BlockSpec((1, D), lambda i: (0, 0)),
    ],
    out_specs=pl.BlockSpec((BW, 8, D), lambda i: (i, 0, 0)),
    out_shape=jax.ShapeDtypeStruct((XPLANES, 8, D), jnp.float32),
)


def kernel(x, edge_index, W1, b1, W2, b2, Wo, bo):
    ei3 = jnp.swapaxes(edge_index.reshape(2, CH, C), 0, 1)
    zeros16 = jnp.zeros((NP, H), jnp.float32)
    ones2d = jnp.ones((C, H), jnp.float32)

    degp = _deg_kernel(ei3, zeros16, ones2d)
    degw = degp.reshape(NC, NWIDE, 128)

    x3 = x.reshape(XPLANES, 8, D)
    w2k = jnp.kron(jnp.eye(8, dtype=jnp.float32), W2)
    b1w = jnp.tile(b1, 8).reshape(1, 128)
    b2w = jnp.tile(b2, 8).reshape(1, 128)

    xw1 = _mm1(x3, W1)
    g1w = _ew1(xw1, degw)
    sp1 = _scatter_kernel(g1w.reshape(NP, H), ei3, zeros16)
    g2w = _l2(sp1.reshape(NC, NWIDE, 128), g1w, degw, w2k, b1w)
    sp2 = _scatter_kernel(g2w.reshape(NP, H), ei3, zeros16)
    out3 = _l3(sp2.reshape(NC, NWIDE, 128), g2w, degw, Wo, b2w, bo.reshape(1, D))
    return out3.reshape(N, D)


# speedup vs baseline: 1.0786x; 1.0786x over previous
import functools

import jax
import jax.numpy as jnp
from jax import lax
from jax.experimental import pallas as pl
from jax.experimental.pallas import tpu as pltpu
from jax.experimental.pallas import tpu_sc as plsc

N = 10000
E = 160000
D = 256
H = 16

NC = 2
NS = 16
NW = NC * NS

C = 128
CH = E // C
K = CH // NW
KMAX = K + 1
HALF = 20
NP = 10240
ACC_ROWS = NP

ROWS_PER_SUB = NP // NS

NWIDE = NP // 8
XPLANES = N // 8

_MESH = plsc.VectorSubcoreMesh(
    core_axis_name="c", subcore_axis_name="s", num_cores=NC, num_subcores=NS
)

_SC_PARAMS = pltpu.CompilerParams(use_tc_tiling_on_sc=False)


def _worker_chunks(c, s):
    w = s * NC + c
    kb = jnp.where(w < CH - K * NW, KMAX, K)
    return w, kb


@functools.partial(
    pl.kernel,
    out_type=jax.ShapeDtypeStruct((NC, NP, H), jnp.float32),
    mesh=_MESH,
    compiler_params=_SC_PARAMS,
    scratch_types=[
        pltpu.VMEM((KMAX, 2, C), jnp.int32),
        pltpu.VMEM((C, H), jnp.float32),
        pltpu.VMEM((ROWS_PER_SUB, H), jnp.float32),
        pltpu.VMEM_SHARED((ACC_ROWS, H), jnp.float32),
        pltpu.SemaphoreType.DMA,
    ],
)
def _deg_kernel(ei3, zeros16, ones2d, out, eidx, ones_v, stage, dacc, lsem):
    c = lax.axis_index("c")
    s = lax.axis_index("s")
    w, kb = _worker_chunks(c, s)
    cp1 = pltpu.async_copy(ei3.at[pl.ds(w * K, K)], eidx.at[pl.ds(0, K)], lsem)
    cp2 = pltpu.async_copy(ones2d, ones_v, lsem)

    @pl.when(w < CH - K * NW)
    def _():
        pltpu.sync_copy(ei3.at[pl.ds(K * NW + w, 1)], eidx.at[pl.ds(K, 1)])

    pltpu.sync_copy(zeros16.at[pl.ds(s * ROWS_PER_SUB, ROWS_PER_SUB)], stage)
    pltpu.sync_copy(stage, dacc.at[pl.ds(s * ROWS_PER_SUB, ROWS_PER_SUB)])
    cp1.wait()
    cp2.wait()
    plsc.subcore_barrier()

    def body(j, carry):
        pltpu.sync_copy(ones_v, dacc.at[eidx.at[j].at[1]], add=True)
        return carry

    lax.fori_loop(0, kb, body, 0)
    plsc.subcore_barrier()
    pltpu.sync_copy(dacc.at[pl.ds(s * ROWS_PER_SUB, ROWS_PER_SUB)], stage)
    pltpu.sync_copy(stage, out.at[c].at[pl.ds(s * ROWS_PER_SUB, ROWS_PER_SUB)])


@functools.partial(
    pl.kernel,
    out_type=jax.ShapeDtypeStruct((NC, NP, H), jnp.float32),
    mesh=_MESH,
    compiler_params=_SC_PARAMS,
    scratch_types=[
        pltpu.VMEM((KMAX, 2, C), jnp.int32),
        pltpu.VMEM((KMAX, C, H), jnp.float32),
        pltpu.VMEM((ROWS_PER_SUB, H), jnp.float32),
        pltpu.VMEM_SHARED((ACC_ROWS, H), jnp.float32),
        pltpu.SemaphoreType.DMA,
        pltpu.SemaphoreType.DMA,
        pltpu.SemaphoreType.DMA,
        pltpu.SemaphoreType.DMA,
    ],
)
def _scatter_kernel(g, ei3, zeros16, out,
                    eidx, rows, stage, acc, lsem, gsem_a, gsem_b, ssem):
    c = lax.axis_index("c")
    s = lax.axis_index("s")
    w, kb = _worker_chunks(c, s)
    cp1 = pltpu.async_copy(ei3.at[pl.ds(w * K, K)], eidx.at[pl.ds(0, K)], lsem)

    @pl.when(w < CH - K * NW)
    def _():
        pltpu.sync_copy(ei3.at[pl.ds(K * NW + w, 1)], eidx.at[pl.ds(K, 1)])

    cp1.wait()

    def fire_a(j, carry):
        pltpu.async_copy(g.at[eidx.at[j].at[0]], rows.at[j], gsem_a)
        return carry

    def fire_b(j, carry):
        pltpu.async_copy(g.at[eidx.at[j].at[0]], rows.at[j], gsem_b)
        return carry

    lax.fori_loop(0, HALF, fire_a, 0)
    lax.fori_loop(HALF, kb, fire_b, 0)

    pltpu.sync_copy(zeros16.at[pl.ds(s * ROWS_PER_SUB, ROWS_PER_SUB)], stage)
    pltpu.sync_copy(stage, acc.at[pl.ds(s * ROWS_PER_SUB, ROWS_PER_SUB)])
    plsc.subcore_barrier()

    def gdrain_a(j, carry):
        pltpu.make_async_copy(g.at[eidx.at[j].at[0]], rows.at[j], gsem_a).wait()
        return carry

    def gdrain_b(j, carry):
        pltpu.make_async_copy(g.at[eidx.at[j].at[0]], rows.at[j], gsem_b).wait()
        return carry

    def sfire(j, carry):
        pltpu.async_copy(rows.at[j], acc.at[eidx.at[j].at[1]], ssem, add=True)
        return carry

    def sdrain(j, carry):
        pltpu.make_async_copy(rows.at[j], acc.at[eidx.at[j].at[1]], ssem).wait()
        return carry

    lax.fori_loop(0, HALF, gdrain_a, 0)
    lax.fori_loop(0, HALF, sfire, 0)
    lax.fori_loop(HALF, kb, gdrain_b, 0)
    lax.fori_loop(HALF, kb, sfire, 0)
    lax.fori_loop(0, kb, sdrain, 0)
    plsc.subcore_barrier()
    pltpu.sync_copy(acc.at[pl.ds(s * ROWS_PER_SUB, ROWS_PER_SUB)], stage)
    pltpu.sync_copy(stage, out.at[c].at[pl.ds(s * ROWS_PER_SUB, ROWS_PER_SUB)])


BW = 128
GRID = NWIDE // BW
BW2 = 256
GRID2 = NWIDE // BW2


def _dinv_of(degw_ref):
    return lax.rsqrt(degw_ref[0] + degw_ref[1] + 1.0)


def _mm1_body(x3_ref, w_ref, o_ref):
    w = w_ref[...]
    parts = [
        jnp.dot(x3_ref[:, k, :], w, preferred_element_type=jnp.float32)
        for k in range(8)
    ]
    o_ref[...] = jnp.concatenate(parts, axis=1)


_mm1 = pl.pallas_call(
    _mm1_body,
    grid=(GRID,),
    in_specs=[
        pl.BlockSpec((BW, 8, D), lambda i: (i, 0, 0)),
        pl.BlockSpec((D, H), lambda i: (0, 0)),
    ],
    out_specs=pl.BlockSpec((BW, 128), lambda i: (i, 0)),
    out_shape=jax.ShapeDtypeStruct((NWIDE, 128), jnp.float32),
)


def _ew1_body(xw_ref, degw_ref, o_ref):
    o_ref[...] = _dinv_of(degw_ref) * xw_ref[...]


_ew1 = pl.pallas_call(
    _ew1_body,
    grid=(GRID2,),
    in_specs=[
        pl.BlockSpec((BW2, 128), lambda i: (i, 0)),
        pl.BlockSpec((NC, BW2, 128), lambda i: (0, i, 0)),
    ],
    out_specs=pl.BlockSpec((BW2, 128), lambda i: (i, 0)),
    out_shape=jax.ShapeDtypeStruct((NWIDE, 128), jnp.float32),
)


def _l2_body(sp_ref, g_ref, degw_ref, w2k_ref, b1w_ref, o_ref):
    dinv = _dinv_of(degw_ref)
    agg = sp_ref[0] + sp_ref[1] + g_ref[...]
    h1 = jnp.maximum(dinv * agg + b1w_ref[...], 0.0)
    o_ref[...] = dinv * jnp.dot(h1, w2k_ref[...], preferred_element_type=jnp.float32)


_l2 = pl.pallas_call(
    _l2_body,
    grid=(GRID2,),
    in_specs=[
        pl.BlockSpec((NC, BW2, 128), lambda i: (0, i, 0)),
        pl.BlockSpec((BW2, 128), lambda i: (i, 0)),
        pl.BlockSpec((NC, BW2, 128), lambda i: (0, i, 0)),
        pl.BlockSpec((128, 128), lambda i: (0, 0)),
        pl.BlockSpec((1, 128), lambda i: (0, 0)),
    ],
    out_specs=pl.BlockSpec((BW2, 128), lambda i: (i, 0)),
    out_shape=jax.ShapeDtypeStruct((NWIDE, 128), jnp.float32),
)


def _l3_body(sp_ref, g_ref, degw_ref, wo_ref, b2w_ref, bo_ref, o_ref):
    dinv = _dinv_of(degw_ref)
    agg = sp_ref[0] + sp_ref[1] + g_ref[...]
    h2 = jnp.maximum(dinv * agg + b2w_ref[...], 0.0)
    wo = wo_ref[...]
    bo = bo_ref[...]
    for k in range(8):
        hk = h2[:, 16 * k:16 * (k + 1)]
        zk = jnp.dot(hk, wo, preferred_element_type=jnp.float32) + bo
        o_ref[:, k, :] = jax.nn.sigmoid(zk)


_l3 = pl.pallas_call(
    _l3_body,
    grid=(GRID,),
    in_specs=[
        pl.BlockSpec((NC, BW, 128), lambda i: (0, i, 0)),
        pl.BlockSpec((BW, 128), lambda i: (i, 0)),
        pl.BlockSpec((NC, BW, 128), lambda i: (0, i, 0)),
        pl.BlockSpec((H, D), lambda i: (0, 0)),
        pl.BlockSpec((1, 128), lambda i: (0, 0)),
        pl.BlockSpec((1, D), lambda i: (0, 0)),
    ],
    out_specs=pl.BlockSpec((BW, 8, D), lambda i: (i, 0, 0)),
    out_shape=jax.ShapeDtypeStruct((XPLANES, 8, D), jnp.float32),
)


def kernel(x, edge_index, W1, b1, W2, b2, Wo, bo):
    ei3 = jnp.swapaxes(edge_index.reshape(2, CH, C), 0, 1)
    zeros16 = jnp.zeros((NP, H), jnp.float32)
    ones2d = jnp.ones((C, H), jnp.float32)

    degp = _deg_kernel(ei3, zeros16, ones2d)
    degw = degp.reshape(NC, NWIDE, 128)

    x3 = x.reshape(XPLANES, 8, D)
    w2k = jnp.kron(jnp.eye(8, dtype=jnp.float32), W2)
    b1w = jnp.tile(b1, 8).reshape(1, 128)
    b2w = jnp.tile(b2, 8).reshape(1, 128)

    xw1 = _mm1(x3, W1)
    g1w = _ew1(xw1, degw)
    sp1 = _scatter_kernel(g1w.reshape(NP, H), ei3, zeros16)
    g2w = _l2(sp1.reshape(NC, NWIDE, 128), g1w, degw, w2k, b1w)
    sp2 = _scatter_kernel(g2w.reshape(NP, H), ei3, zeros16)
    out3 = _l3(sp2.reshape(NC, NWIDE, 128), g2w, degw, Wo, b2w, bo.reshape(1, D))
    return out3.reshape(N, D)
